# Initial kernel scaffold; baseline (speedup 1.0000x reference)
import jax
import jax.numpy as jnp
from jax.experimental import pallas as pl


def kernel(x, edge_index, W1, b1, W2, b2):
    raise NotImplementedError("write your pallas kernel here")


# speedup vs baseline: 11.1019x; 11.1019x over previous
import jax
import jax.numpy as jnp
from jax import lax
from jax.experimental import pallas as pl
from jax.experimental.pallas import tpu as pltpu
from jax.experimental.pallas import tpu_sc as plsc

N = 10000
D = 128
E = 320000

NC = 2
NS = 16
NW = NC * NS
EPT = E // NW

K = 128
NB = 80
EPADT = NB * K
CH = 8
NCH = NB // CH

NPAD = 10240
TRASH = N
DPT = NPAD // NS
ZR = 128

_f32 = jnp.float32
_i32 = jnp.int32

_mesh = plsc.VectorSubcoreMesh(
    core_axis_name="c", subcore_axis_name="s", num_cores=NC, num_subcores=NS)


def _deg_body(colb_hbm, deg_out, col_v, ones_v, zrow_v, deg_sh):
    c = lax.axis_index("c")
    s = lax.axis_index("s")
    wid = c * NS + s
    pltpu.sync_copy(colb_hbm.at[wid], col_v)
    one16 = jnp.ones((16,), _f32)
    zero16 = jnp.zeros((16,), _f32)
    for t in range(K // 16):
        ones_v[pl.ds(t * 16, 16)] = one16
    for t in range(DPT // 16):
        zrow_v[pl.ds(t * 16, 16)] = zero16
    pltpu.sync_copy(zrow_v, deg_sh.at[pl.ds(s * DPT, DPT)])
    plsc.subcore_barrier()

    def step(j, carry):
        pltpu.sync_copy(ones_v, deg_sh.at[col_v.at[j]], add=True)
        return carry

    lax.fori_loop(0, NB, step, 0)
    plsc.subcore_barrier()
    pltpu.sync_copy(deg_sh.at[pl.ds(s * DPT, DPT)],
                    deg_out.at[c, pl.ds(s * DPT, DPT)])


_deg = pl.kernel(
    _deg_body,
    out_type=jax.ShapeDtypeStruct((NC, NPAD), _f32),
    mesh=_mesh,
    scratch_types=[
        pltpu.VMEM((NB, K), _i32),
        pltpu.VMEM((K,), _f32),
        pltpu.VMEM((DPT,), _f32),
        pltpu.VMEM_SHARED((NPAD,), _f32),
    ],
)


def _prop_body(g_hbm, rowb_hbm, colb_hbm, zeros_hbm, acc_out,
               rbuf0, rbuf1, cbuf0, cbuf1, buf0, buf1,
               rsem, csem, sem0, sem1, zsem, acc_sh):
    c = lax.axis_index("c")
    s = lax.axis_index("s")
    wid = c * NS + s
    rbufs = (rbuf0, rbuf1)
    cbufs = (cbuf0, cbuf1)
    bufs = (buf0, buf1)
    gsems = (sem0, sem1)

    for t in range(DPT // ZR):
        pltpu.async_copy(zeros_hbm,
                         acc_sh.at[pl.ds(s * DPT + t * ZR, ZR)], zsem)
    pltpu.sync_copy(rowb_hbm.at[wid, pl.ds(0, CH)], rbuf0)
    pltpu.sync_copy(colb_hbm.at[wid, pl.ds(0, CH)], cbuf0)
    pltpu.async_copy(rowb_hbm.at[wid, pl.ds(CH, CH)], rbuf1, rsem)
    pltpu.async_copy(colb_hbm.at[wid, pl.ds(CH, CH)], cbuf1, csem)
    for t in range(DPT // ZR):
        pltpu.make_async_copy(
            zeros_hbm, acc_sh.at[pl.ds(s * DPT + t * ZR, ZR)], zsem).wait()
    plsc.subcore_barrier()

    pltpu.async_copy(g_hbm.at[rbuf0.at[0]], buf0, sem0)
    pltpu.async_copy(g_hbm.at[rbuf0.at[1]], buf1, sem1)

    def qstep(qq, carry):
        for qp in range(2):
            q = qq * 2 + qp
            for i in range(CH):
                par = i % 2
                pltpu.make_async_copy(
                    g_hbm.at[rbufs[qp].at[i]], bufs[par], gsems[par]).wait()
                pltpu.sync_copy(bufs[par], acc_sh.at[cbufs[qp].at[i]],
                                add=True)
                if i == CH - 2:
                    @pl.when(q + 1 < NCH)
                    def _():
                        pltpu.make_async_copy(
                            rowb_hbm.at[wid, pl.ds((q + 1) * CH, CH)],
                            rbufs[1 - qp], rsem).wait()
                        pltpu.make_async_copy(
                            colb_hbm.at[wid, pl.ds((q + 1) * CH, CH)],
                            cbufs[1 - qp], csem).wait()
                if i < CH - 2:
                    pltpu.async_copy(
                        g_hbm.at[rbufs[qp].at[i + 2]], bufs[par], gsems[par])
                else:
                    @pl.when(q + 1 < NCH)
                    def _():
                        pltpu.async_copy(
                            g_hbm.at[rbufs[1 - qp].at[i - (CH - 2)]],
                            bufs[par], gsems[par])
            @pl.when(q + 2 < NCH)
            def _():
                pltpu.async_copy(
                    rowb_hbm.at[wid, pl.ds((q + 2) * CH, CH)],
                    rbufs[qp], rsem)
                pltpu.async_copy(
                    colb_hbm.at[wid, pl.ds((q + 2) * CH, CH)],
                    cbufs[qp], csem)
        return carry

    lax.fori_loop(0, NCH // 2, qstep, 0)
    plsc.subcore_barrier()
    pltpu.sync_copy(acc_sh.at[pl.ds(s * DPT, DPT)],
                    acc_out.at[c, pl.ds(s * DPT, DPT)])


_prop = pl.kernel(
    _prop_body,
    out_type=jax.ShapeDtypeStruct((NC, NPAD, D), _f32),
    mesh=_mesh,
    scratch_types=[
        pltpu.VMEM((CH, K), _i32),
        pltpu.VMEM((CH, K), _i32),
        pltpu.VMEM((CH, K), _i32),
        pltpu.VMEM((CH, K), _i32),
        pltpu.VMEM((K, D), _f32),
        pltpu.VMEM((K, D), _f32),
        pltpu.SemaphoreType.DMA,
        pltpu.SemaphoreType.DMA,
        pltpu.SemaphoreType.DMA,
        pltpu.SemaphoreType.DMA,
        pltpu.SemaphoreType.DMA,
        pltpu.VMEM_SHARED((NPAD, D), _f32),
    ],
)


def _pre_body(x_ref, w_ref, d_ref, g_ref, dinv_ref):
    deg = d_ref[0] + d_ref[1] + 1.0
    dinv = lax.rsqrt(deg)
    dinv_ref[...] = dinv
    g_ref[...] = jnp.dot(x_ref[...], w_ref[...],
                         preferred_element_type=_f32) * dinv


_pre = pl.pallas_call(
    _pre_body,
    out_shape=(jax.ShapeDtypeStruct((N, D), _f32),
               jax.ShapeDtypeStruct((N, 1), _f32)),
)


def _mid_body(a_ref, g1_ref, dinv_ref, b_ref, w_ref, g2_ref):
    dinv = dinv_ref[...]
    h = (a_ref[0] + a_ref[1] + g1_ref[...]) * dinv + b_ref[...]
    h = jnp.maximum(h, 0.0)
    g2_ref[...] = jnp.dot(h, w_ref[...], preferred_element_type=_f32) * dinv


_mid = pl.pallas_call(
    _mid_body,
    out_shape=jax.ShapeDtypeStruct((N, D), _f32),
)


def _post_body(a_ref, g2_ref, dinv_ref, b_ref, out_ref):
    out_ref[...] = (a_ref[0] + a_ref[1] + g2_ref[...]) * dinv_ref[...] \
        + b_ref[...]


_post = pl.pallas_call(
    _post_body,
    out_shape=jax.ShapeDtypeStruct((N, D), _f32),
)


def kernel(x, edge_index, W1, b1, W2, b2):
    ei = edge_index.astype(_i32)
    r2 = ei[0].reshape(NW, EPT)
    c2 = ei[1].reshape(NW, EPT)
    rowb = jnp.zeros((NW, EPADT), _i32).at[:, :EPT].set(r2) \
        .reshape(NW, NB, K)
    colb = jnp.full((NW, EPADT), TRASH, _i32).at[:, :EPT].set(c2) \
        .reshape(NW, NB, K)
    zeros = jnp.zeros((ZR, D), _f32)

    deg_raw = _deg(colb)
    d3 = deg_raw[:, :N, None]
    g1, dinv = _pre(x, W1, d3)
    acc1 = _prop(g1, rowb, colb, zeros)[:, :N]
    g2 = _mid(acc1, g1, dinv, b1.reshape(1, D), W2)
    acc2 = _prop(g2, rowb, colb, zeros)[:, :N]
    return _post(acc2, g2, dinv, b2.reshape(1, D))
